# Initial kernel scaffold; baseline (speedup 1.0000x reference)
import jax
import jax.numpy as jnp
from jax.experimental import pallas as pl


def kernel(x, edge_index, W1, b1, W2, b2, L1_W, L1_b, L2_W, L2_b):
    raise NotImplementedError("write your pallas kernel here")


# speedup vs baseline: 23.6514x; 23.6514x over previous
import functools

import jax
import jax.numpy as jnp
from jax import lax
from jax.experimental import pallas as pl
from jax.experimental.pallas import tpu as pltpu
from jax.experimental.pallas import tpu_sc as plsc

N = 10000
NP = 10240
D = 128
E = 320000
NW = 32
CH = 80
CL = 128
EP = NW * CH * CL
RB = NP // 16

_f32 = jnp.float32


def _sc_mesh():
    return plsc.VectorSubcoreMesh(core_axis_name="c", subcore_axis_name="s")


def _sc_degree(dst_idx, zeros_row, ones_row):
    @functools.partial(
        pl.kernel,
        out_type=jax.ShapeDtypeStruct((2, NP), _f32),
        mesh=_sc_mesh(),
        scratch_types=[
            pltpu.VMEM((CH, CL), jnp.int32),
            pltpu.VMEM((CL,), _f32),
            pltpu.VMEM_SHARED((NP,), _f32),
            pltpu.SemaphoreType.DMA,
            pltpu.SemaphoreType.DMA,
        ],
    )
    def k(dst_hbm, zeros_hbm, ones_hbm, out_hbm, idx_d, ones_v, acc, s0, s1):
        cid = lax.axis_index("c")
        sid = lax.axis_index("s")
        wid = sid * 2 + cid
        pltpu.sync_copy(dst_hbm.at[wid], idx_d)
        pltpu.sync_copy(ones_hbm, ones_v)
        pltpu.sync_copy(zeros_hbm, acc.at[pl.ds(sid * RB, RB)])
        plsc.subcore_barrier()

        @pl.loop(0, CH // 2)
        def _(j):
            d0 = pltpu.async_copy(ones_v, acc.at[idx_d.at[2 * j]], s0, add=True)
            d1 = pltpu.async_copy(ones_v, acc.at[idx_d.at[2 * j + 1]], s1, add=True)
            d0.wait()
            d1.wait()

        plsc.subcore_barrier()
        pltpu.sync_copy(acc.at[pl.ds(sid * RB, RB)],
                        out_hbm.at[cid, pl.ds(sid * RB, RB)])

    return k(dst_idx, zeros_row, ones_row)


def _sc_gather_scatter(table, src_idx, dst_idx, zeros_blk):
    @functools.partial(
        pl.kernel,
        out_type=jax.ShapeDtypeStruct((2, NP, D), _f32),
        mesh=_sc_mesh(),
        scratch_types=[
            pltpu.VMEM((CH // 2, CL), jnp.int32),
            pltpu.VMEM((CH // 2, CL), jnp.int32),
            pltpu.VMEM((CL, D), _f32),
            pltpu.VMEM((CL, D), _f32),
            pltpu.VMEM_SHARED((NP, D), _f32),
            pltpu.SemaphoreType.DMA,
            pltpu.SemaphoreType.DMA,
        ],
    )
    def k(tab_hbm, src_hbm, dst_hbm, zeros_hbm, out_hbm,
          idx_s, idx_d, buf0, buf1, acc, s0, s1):
        cid = lax.axis_index("c")
        sid = lax.axis_index("s")
        wid = sid * 2 + cid
        pltpu.sync_copy(zeros_hbm, acc.at[pl.ds(sid * RB, RB)])
        plsc.subcore_barrier()

        for h in range(2):
            pltpu.sync_copy(src_hbm.at[wid, pl.ds(h * (CH // 2), CH // 2)], idx_s)
            pltpu.sync_copy(dst_hbm.at[wid, pl.ds(h * (CH // 2), CH // 2)], idx_d)

            @pl.loop(0, CH // 4)
            def _(j):
                g0 = pltpu.async_copy(tab_hbm.at[idx_s.at[2 * j]], buf0, s0)
                g1 = pltpu.async_copy(tab_hbm.at[idx_s.at[2 * j + 1]], buf1, s1)
                g0.wait()
                pltpu.sync_copy(buf0, acc.at[idx_d.at[2 * j]], add=True)
                g1.wait()
                pltpu.sync_copy(buf1, acc.at[idx_d.at[2 * j + 1]], add=True)

        plsc.subcore_barrier()
        pltpu.sync_copy(acc.at[pl.ds(sid * RB, RB)],
                        out_hbm.at[cid, pl.ds(sid * RB, RB)])

    return k(table, src_idx, dst_idx, zeros_blk)


def _tc_layer1(x, degs, W1):
    def body(x_ref, d_ref, w_ref, h_ref, dinv_ref):
        din = lax.rsqrt(d_ref[0] + d_ref[1] + 1.0)
        h = jnp.dot(x_ref[...], w_ref[...], preferred_element_type=_f32)
        h_ref[...] = h * din
        dinv_ref[...] = din

    grid = NP // 512
    return pl.pallas_call(
        body,
        grid=(grid,),
        in_specs=[
            pl.BlockSpec((512, D), lambda i: (i, 0)),
            pl.BlockSpec((2, 512, 1), lambda i: (0, i, 0)),
            pl.BlockSpec((D, D), lambda i: (0, 0)),
        ],
        out_specs=[
            pl.BlockSpec((512, D), lambda i: (i, 0)),
            pl.BlockSpec((512, 1), lambda i: (i, 0)),
        ],
        out_shape=[
            jax.ShapeDtypeStruct((NP, D), _f32),
            jax.ShapeDtypeStruct((NP, 1), _f32),
        ],
    )(x, degs, W1)


def _tc_layer2(acc, h1p, dinv, b1, W2):
    def body(a_ref, h_ref, d_ref, b_ref, w_ref, o_ref):
        din = d_ref[...]
        s = (a_ref[0] + a_ref[1] + h_ref[...]) * din + b_ref[...]
        h2 = jnp.maximum(s, 0.0)
        o_ref[...] = jnp.dot(h2, w_ref[...], preferred_element_type=_f32) * din

    grid = NP // 512
    return pl.pallas_call(
        body,
        grid=(grid,),
        in_specs=[
            pl.BlockSpec((2, 512, D), lambda i: (0, i, 0)),
            pl.BlockSpec((512, D), lambda i: (i, 0)),
            pl.BlockSpec((512, 1), lambda i: (i, 0)),
            pl.BlockSpec((1, D), lambda i: (0, 0)),
            pl.BlockSpec((D, D), lambda i: (0, 0)),
        ],
        out_specs=pl.BlockSpec((512, D), lambda i: (i, 0)),
        out_shape=jax.ShapeDtypeStruct((NP, D), _f32),
    )(acc, h1p, dinv, b1, W2)


def _tc_head(acc, h2p, dinv, b2, L1_W, L1_b, L2_W, L2_b):
    def body(a_ref, h_ref, d_ref, b_ref, w1_ref, b1_ref, w2_ref, b2_ref, o_ref):
        s = (a_ref[0] + a_ref[1] + h_ref[...]) * d_ref[...] + b_ref[...]
        h3 = jnp.maximum(s, 0.0)
        t = jnp.dot(h3, w1_ref[...], preferred_element_type=_f32) + b1_ref[...]
        o_ref[...] = jnp.dot(t, w2_ref[...], preferred_element_type=_f32) + b2_ref[...]

    grid = NP // 512
    return pl.pallas_call(
        body,
        grid=(grid,),
        in_specs=[
            pl.BlockSpec((2, 512, D), lambda i: (0, i, 0)),
            pl.BlockSpec((512, D), lambda i: (i, 0)),
            pl.BlockSpec((512, 1), lambda i: (i, 0)),
            pl.BlockSpec((1, D), lambda i: (0, 0)),
            pl.BlockSpec((D, 4), lambda i: (0, 0)),
            pl.BlockSpec((1, 4), lambda i: (0, 0)),
            pl.BlockSpec((4, 16), lambda i: (0, 0)),
            pl.BlockSpec((1, 16), lambda i: (0, 0)),
        ],
        out_specs=pl.BlockSpec((512, 16), lambda i: (i, 0)),
        out_shape=jax.ShapeDtypeStruct((NP, 16), _f32),
    )(acc, h2p, dinv, b2, L1_W, L1_b, L2_W, L2_b)


def kernel(x, edge_index, W1, b1, W2, b2, L1_W, L1_b, L2_W, L2_b):
    x_pad = jnp.pad(x, ((0, NP - N), (0, 0)))
    pad_i = jnp.arange(EP - E, dtype=jnp.int32)
    pad_row = N + (pad_i % (NP - N))
    src = jnp.concatenate([edge_index[0], pad_row]).reshape(NW, CH, CL)
    dst = jnp.concatenate([edge_index[1], pad_row]).reshape(NW, CH, CL)
    zeros_row = jnp.zeros((RB,), _f32)
    ones_row = jnp.ones((CL,), _f32)
    zeros_blk = jnp.zeros((RB, D), _f32)

    degs = _sc_degree(dst, zeros_row, ones_row)
    degs = degs.reshape(2, NP, 1)

    h1p, dinv = _tc_layer1(x_pad, degs, W1)
    acc1 = _sc_gather_scatter(h1p, src, dst, zeros_blk)
    h2p = _tc_layer2(acc1, h1p, dinv, b1.reshape(1, D), W2)
    acc2 = _sc_gather_scatter(h2p, src, dst, zeros_blk)
    out = _tc_head(acc2, h2p, dinv, b2.reshape(1, D),
                   L1_W, L1_b.reshape(1, 4), L2_W, L2_b.reshape(1, 16))
    return out[:N]


# speedup vs baseline: 24.7420x; 1.0461x over previous
import functools

import jax
import jax.numpy as jnp
from jax import lax
from jax.experimental import pallas as pl
from jax.experimental.pallas import tpu as pltpu
from jax.experimental.pallas import tpu_sc as plsc

N = 10000
NP = 10240
D = 128
E = 320000
NW = 32
CH = 80
CL = 128
EP = NW * CH * CL
RB = NP // 16

_f32 = jnp.float32


def _sc_mesh():
    return plsc.VectorSubcoreMesh(core_axis_name="c", subcore_axis_name="s")


def _sc_degree(dst_idx, zeros_row, ones_row):
    @functools.partial(
        pl.kernel,
        out_type=jax.ShapeDtypeStruct((2, NP), _f32),
        mesh=_sc_mesh(),
        scratch_types=[
            pltpu.VMEM((CH, CL), jnp.int32),
            pltpu.VMEM((CL,), _f32),
            pltpu.VMEM_SHARED((NP,), _f32),
            pltpu.SemaphoreType.DMA,
            pltpu.SemaphoreType.DMA,
        ],
    )
    def k(dst_hbm, zeros_hbm, ones_hbm, out_hbm, idx_d, ones_v, acc, s0, s1):
        cid = lax.axis_index("c")
        sid = lax.axis_index("s")
        wid = sid * 2 + cid
        pltpu.sync_copy(dst_hbm.at[wid], idx_d)
        pltpu.sync_copy(ones_hbm, ones_v)
        pltpu.sync_copy(zeros_hbm, acc.at[pl.ds(sid * RB, RB)])
        plsc.subcore_barrier()

        @pl.loop(0, CH // 2)
        def _(j):
            d0 = pltpu.async_copy(ones_v, acc.at[idx_d.at[2 * j]], s0, add=True)
            d1 = pltpu.async_copy(ones_v, acc.at[idx_d.at[2 * j + 1]], s1, add=True)
            d0.wait()
            d1.wait()

        plsc.subcore_barrier()
        pltpu.sync_copy(acc.at[pl.ds(sid * RB, RB)],
                        out_hbm.at[cid, pl.ds(sid * RB, RB)])

    return k(dst_idx, zeros_row, ones_row)


def _sc_gather_scatter(table, src_idx, dst_idx):
    @functools.partial(
        pl.kernel,
        out_type=jax.ShapeDtypeStruct((2, NP, D), _f32),
        mesh=_sc_mesh(),
        scratch_types=[
            pltpu.VMEM((CH // 2, CL), jnp.int32),
            pltpu.VMEM((CH // 2, CL), jnp.int32),
            pltpu.VMEM((CL, D), _f32),
            pltpu.VMEM((CL, D), _f32),
            pltpu.VMEM_SHARED((NP, D), _f32),
            pltpu.SemaphoreType.DMA,
            pltpu.SemaphoreType.DMA,
            pltpu.SemaphoreType.DMA,
            pltpu.SemaphoreType.DMA,
        ],
    )
    def k(tab_hbm, src_hbm, dst_hbm, out_hbm,
          idx_s, idx_d, buf0, buf1, acc, sg0, sg1, ss0, ss1):
        cid = lax.axis_index("c")
        sid = lax.axis_index("s")
        wid = sid * 2 + cid
        HC = CH // 2

        @pl.loop(0, CL)
        def _(r):
            for i in range(D // 16):
                buf0[r, pl.ds(i * 16, 16)] = jnp.zeros((16,), _f32)

        for z in range(RB // CL):
            pltpu.sync_copy(buf0, acc.at[pl.ds(sid * RB + z * CL, CL)])
        plsc.subcore_barrier()

        for h in range(2):
            pltpu.sync_copy(src_hbm.at[wid, pl.ds(h * HC, HC)], idx_s)
            pltpu.sync_copy(dst_hbm.at[wid, pl.ds(h * HC, HC)], idx_d)
            pltpu.async_copy(tab_hbm.at[idx_s.at[0]], buf0, sg0)
            pltpu.async_copy(tab_hbm.at[idx_s.at[1]], buf1, sg1)

            @pl.loop(0, HC // 2)
            def _(j):
                pltpu.make_async_copy(tab_hbm.at[idx_s.at[0]], buf0, sg0).wait()
                pltpu.async_copy(buf0, acc.at[idx_d.at[2 * j]], ss0, add=True)
                pltpu.make_async_copy(tab_hbm.at[idx_s.at[0]], buf1, sg1).wait()
                pltpu.async_copy(buf1, acc.at[idx_d.at[2 * j + 1]], ss1, add=True)

                @pl.when(j < HC // 2 - 1)
                def _():
                    pltpu.make_async_copy(buf0, acc.at[idx_d.at[0]], ss0).wait()
                    pltpu.async_copy(tab_hbm.at[idx_s.at[2 * j + 2]], buf0, sg0)
                    pltpu.make_async_copy(buf1, acc.at[idx_d.at[0]], ss1).wait()
                    pltpu.async_copy(tab_hbm.at[idx_s.at[2 * j + 3]], buf1, sg1)

            pltpu.make_async_copy(buf0, acc.at[idx_d.at[0]], ss0).wait()
            pltpu.make_async_copy(buf1, acc.at[idx_d.at[0]], ss1).wait()

        plsc.subcore_barrier()
        pltpu.sync_copy(acc.at[pl.ds(sid * RB, RB)],
                        out_hbm.at[cid, pl.ds(sid * RB, RB)])

    return k(table, src_idx, dst_idx)


def _tc_layer1(x, degs, W1):
    def body(x_ref, d_ref, w_ref, h_ref, dinv_ref):
        din = lax.rsqrt(d_ref[0] + d_ref[1] + 1.0)
        h = jnp.dot(x_ref[...], w_ref[...], preferred_element_type=_f32)
        h_ref[...] = h * din
        dinv_ref[...] = din

    grid = NP // 512
    return pl.pallas_call(
        body,
        grid=(grid,),
        in_specs=[
            pl.BlockSpec((512, D), lambda i: (i, 0)),
            pl.BlockSpec((2, 512, 1), lambda i: (0, i, 0)),
            pl.BlockSpec((D, D), lambda i: (0, 0)),
        ],
        out_specs=[
            pl.BlockSpec((512, D), lambda i: (i, 0)),
            pl.BlockSpec((512, 1), lambda i: (i, 0)),
        ],
        out_shape=[
            jax.ShapeDtypeStruct((NP, D), _f32),
            jax.ShapeDtypeStruct((NP, 1), _f32),
        ],
    )(x, degs, W1)


def _tc_layer2(acc, h1p, dinv, b1, W2):
    def body(a_ref, h_ref, d_ref, b_ref, w_ref, o_ref):
        din = d_ref[...]
        s = (a_ref[0] + a_ref[1] + h_ref[...]) * din + b_ref[...]
        h2 = jnp.maximum(s, 0.0)
        o_ref[...] = jnp.dot(h2, w_ref[...], preferred_element_type=_f32) * din

    grid = NP // 512
    return pl.pallas_call(
        body,
        grid=(grid,),
        in_specs=[
            pl.BlockSpec((2, 512, D), lambda i: (0, i, 0)),
            pl.BlockSpec((512, D), lambda i: (i, 0)),
            pl.BlockSpec((512, 1), lambda i: (i, 0)),
            pl.BlockSpec((1, D), lambda i: (0, 0)),
            pl.BlockSpec((D, D), lambda i: (0, 0)),
        ],
        out_specs=pl.BlockSpec((512, D), lambda i: (i, 0)),
        out_shape=jax.ShapeDtypeStruct((NP, D), _f32),
    )(acc, h1p, dinv, b1, W2)


def _tc_head(acc, h2p, dinv, b2, L1_W, L1_b, L2_W, L2_b):
    def body(a_ref, h_ref, d_ref, b_ref, w1_ref, b1_ref, w2_ref, b2_ref, o_ref):
        s = (a_ref[0] + a_ref[1] + h_ref[...]) * d_ref[...] + b_ref[...]
        h3 = jnp.maximum(s, 0.0)
        t = jnp.dot(h3, w1_ref[...], preferred_element_type=_f32) + b1_ref[...]
        o_ref[...] = jnp.dot(t, w2_ref[...], preferred_element_type=_f32) + b2_ref[...]

    grid = NP // 512
    return pl.pallas_call(
        body,
        grid=(grid,),
        in_specs=[
            pl.BlockSpec((2, 512, D), lambda i: (0, i, 0)),
            pl.BlockSpec((512, D), lambda i: (i, 0)),
            pl.BlockSpec((512, 1), lambda i: (i, 0)),
            pl.BlockSpec((1, D), lambda i: (0, 0)),
            pl.BlockSpec((D, 4), lambda i: (0, 0)),
            pl.BlockSpec((1, 4), lambda i: (0, 0)),
            pl.BlockSpec((4, 16), lambda i: (0, 0)),
            pl.BlockSpec((1, 16), lambda i: (0, 0)),
        ],
        out_specs=pl.BlockSpec((512, 16), lambda i: (i, 0)),
        out_shape=jax.ShapeDtypeStruct((NP, 16), _f32),
    )(acc, h2p, dinv, b2, L1_W, L1_b, L2_W, L2_b)


def kernel(x, edge_index, W1, b1, W2, b2, L1_W, L1_b, L2_W, L2_b):
    x_pad = jnp.pad(x, ((0, NP - N), (0, 0)))
    pad_i = jnp.arange(EP - E, dtype=jnp.int32)
    pad_row = N + (pad_i % (NP - N))
    src = jnp.concatenate([edge_index[0], pad_row]).reshape(NW, CH, CL)
    dst = jnp.concatenate([edge_index[1], pad_row]).reshape(NW, CH, CL)
    zeros_row = jnp.zeros((RB,), _f32)
    ones_row = jnp.ones((CL,), _f32)

    degs = _sc_degree(dst, zeros_row, ones_row)
    degs = degs.reshape(2, NP, 1)

    h1p, dinv = _tc_layer1(x_pad, degs, W1)
    acc1 = _sc_gather_scatter(h1p, src, dst)
    h2p = _tc_layer2(acc1, h1p, dinv, b1.reshape(1, D), W2)
    acc2 = _sc_gather_scatter(h2p, src, dst)
    out = _tc_head(acc2, h2p, dinv, b2.reshape(1, D),
                   L1_W, L1_b.reshape(1, 4), L2_W, L2_b.reshape(1, 16))
    return out[:N]


# speedup vs baseline: 30.4613x; 1.2312x over previous
import functools

import jax
import jax.numpy as jnp
from jax import lax
from jax.experimental import pallas as pl
from jax.experimental.pallas import tpu as pltpu
from jax.experimental.pallas import tpu_sc as plsc

N = 10000
NP = 10240
D = 128
E = 320000
NW = 32
CH = 80
CL = 128
EP = NW * CH * CL
RB = NP // 16

_f32 = jnp.float32
_bf16 = jnp.bfloat16


def _sc_mesh():
    return plsc.VectorSubcoreMesh(core_axis_name="c", subcore_axis_name="s")


def _sc_degree(dst_idx, ones_row):
    @functools.partial(
        pl.kernel,
        out_type=jax.ShapeDtypeStruct((2, NP), _f32),
        mesh=_sc_mesh(),
        scratch_types=[
            pltpu.VMEM((CH, CL), jnp.int32),
            pltpu.VMEM((CL,), _f32),
            pltpu.VMEM((RB,), _f32),
            pltpu.VMEM_SHARED((NP,), _f32),
            pltpu.SemaphoreType.DMA,
            pltpu.SemaphoreType.DMA,
        ],
    )
    def k(dst_hbm, ones_hbm, out_hbm, idx_d, ones_v, zero_v, acc, s0, s1):
        cid = lax.axis_index("c")
        sid = lax.axis_index("s")
        wid = sid * 2 + cid
        pltpu.sync_copy(dst_hbm.at[wid], idx_d)
        pltpu.sync_copy(ones_hbm, ones_v)

        @pl.loop(0, RB // 16)
        def _(r):
            zero_v[pl.ds(r * 16, 16)] = jnp.zeros((16,), _f32)

        pltpu.sync_copy(zero_v, acc.at[pl.ds(sid * RB, RB)])
        plsc.subcore_barrier()

        @pl.loop(0, CH // 2)
        def _(j):
            d0 = pltpu.async_copy(ones_v, acc.at[idx_d.at[2 * j]], s0, add=True)
            d1 = pltpu.async_copy(ones_v, acc.at[idx_d.at[2 * j + 1]], s1, add=True)
            d0.wait()
            d1.wait()

        plsc.subcore_barrier()
        pltpu.sync_copy(acc.at[pl.ds(sid * RB, RB)],
                        out_hbm.at[cid, pl.ds(sid * RB, RB)])

    return k(dst_idx, ones_row)


def _sc_gather_scatter(table, src_idx, dst_idx):
    HC = CH // 2

    @functools.partial(
        pl.kernel,
        out_type=jax.ShapeDtypeStruct((2, NP, D), _f32),
        mesh=_sc_mesh(),
        scratch_types=[
            pltpu.VMEM((HC, CL), jnp.int32),
            pltpu.VMEM((HC, CL), jnp.int32),
            pltpu.VMEM((CL, D), _f32),
            pltpu.VMEM((CL, D), _f32),
            pltpu.VMEM_SHARED((NP, D), _f32),
            pltpu.SemaphoreType.DMA,
            pltpu.SemaphoreType.DMA,
            pltpu.SemaphoreType.DMA,
            pltpu.SemaphoreType.DMA,
        ],
    )
    def k(tab_hbm, src_hbm, dst_hbm, out_hbm,
          idx_s, idx_d, fb0, fb1, acc, sg0, sg1, ss0, ss1):
        cid = lax.axis_index("c")
        sid = lax.axis_index("s")
        wid = sid * 2 + cid

        @pl.loop(0, CL)
        def _(r):
            for i in range(D // 16):
                fb0[r, pl.ds(i * 16, 16)] = jnp.zeros((16,), _f32)

        for z in range(RB // CL):
            pltpu.sync_copy(fb0, acc.at[pl.ds(sid * RB + z * CL, CL)])
        plsc.subcore_barrier()

        for h in range(2):
            pltpu.sync_copy(src_hbm.at[wid, pl.ds(h * HC, HC)], idx_s)
            pltpu.sync_copy(dst_hbm.at[wid, pl.ds(h * HC, HC)], idx_d)
            pltpu.async_copy(tab_hbm.at[idx_s.at[0]], fb0, sg0)
            pltpu.async_copy(tab_hbm.at[idx_s.at[1]], fb1, sg1)

            @pl.loop(0, HC // 2)
            def _(j):
                for p, (fb, sg, ss) in enumerate(
                        ((fb0, sg0, ss0), (fb1, sg1, ss1))):
                    pltpu.make_async_copy(tab_hbm.at[idx_s.at[0]], fb, sg).wait()
                    pltpu.async_copy(fb, acc.at[idx_d.at[2 * j + p]], ss, add=True)
                    pltpu.make_async_copy(fb, acc.at[idx_d.at[0]], ss).wait()

                    @pl.when(j < HC // 2 - 1)
                    def _():
                        pltpu.async_copy(tab_hbm.at[idx_s.at[2 * j + 2 + p]], fb, sg)

        plsc.subcore_barrier()
        pltpu.sync_copy(acc.at[pl.ds(sid * RB, RB)],
                        out_hbm.at[cid, pl.ds(sid * RB, RB)])

    return k(table, src_idx, dst_idx)


def _tc_layer1(x, degs, W1):
    def body(x_ref, d_ref, w_ref, h_ref, dinv_ref):
        din = lax.rsqrt(d_ref[0] + d_ref[1] + 1.0)
        h = jnp.dot(x_ref[...], w_ref[...], preferred_element_type=_f32)
        h_ref[...] = h * din
        dinv_ref[...] = din

    grid = NP // 512
    return pl.pallas_call(
        body,
        grid=(grid,),
        in_specs=[
            pl.BlockSpec((512, D), lambda i: (i, 0)),
            pl.BlockSpec((2, 512, 1), lambda i: (0, i, 0)),
            pl.BlockSpec((D, D), lambda i: (0, 0)),
        ],
        out_specs=[
            pl.BlockSpec((512, D), lambda i: (i, 0)),
            pl.BlockSpec((512, 1), lambda i: (i, 0)),
        ],
        out_shape=[
            jax.ShapeDtypeStruct((NP, D), _f32),
            jax.ShapeDtypeStruct((NP, 1), _f32),
        ],
    )(x, degs, W1)


def _tc_layer2(acc, h1p, dinv, b1, W2):
    def body(a_ref, h_ref, d_ref, b_ref, w_ref, o_ref):
        din = d_ref[...]
        s = (a_ref[0] + a_ref[1] + h_ref[...]) * din + b_ref[...]
        h2 = jnp.maximum(s, 0.0)
        o_ref[...] = jnp.dot(h2, w_ref[...], preferred_element_type=_f32) * din

    grid = NP // 512
    return pl.pallas_call(
        body,
        grid=(grid,),
        in_specs=[
            pl.BlockSpec((2, 512, D), lambda i: (0, i, 0)),
            pl.BlockSpec((512, D), lambda i: (i, 0)),
            pl.BlockSpec((512, 1), lambda i: (i, 0)),
            pl.BlockSpec((1, D), lambda i: (0, 0)),
            pl.BlockSpec((D, D), lambda i: (0, 0)),
        ],
        out_specs=pl.BlockSpec((512, D), lambda i: (i, 0)),
        out_shape=jax.ShapeDtypeStruct((NP, D), _f32),
    )(acc, h1p, dinv, b1, W2)


def _tc_head(acc, h2p, dinv, b2, L1_W, L1_b, L2_W, L2_b):
    def body(a_ref, h_ref, d_ref, b_ref, w1_ref, b1_ref, w2_ref, b2_ref, o_ref):
        s = (a_ref[0] + a_ref[1] + h_ref[...]) * d_ref[...] + b_ref[...]
        h3 = jnp.maximum(s, 0.0)
        t = jnp.dot(h3, w1_ref[...], preferred_element_type=_f32) + b1_ref[...]
        o_ref[...] = jnp.dot(t, w2_ref[...], preferred_element_type=_f32) + b2_ref[...]

    grid = NP // 512
    return pl.pallas_call(
        body,
        grid=(grid,),
        in_specs=[
            pl.BlockSpec((2, 512, D), lambda i: (0, i, 0)),
            pl.BlockSpec((512, D), lambda i: (i, 0)),
            pl.BlockSpec((512, 1), lambda i: (i, 0)),
            pl.BlockSpec((1, D), lambda i: (0, 0)),
            pl.BlockSpec((D, 4), lambda i: (0, 0)),
            pl.BlockSpec((1, 4), lambda i: (0, 0)),
            pl.BlockSpec((4, 16), lambda i: (0, 0)),
            pl.BlockSpec((1, 16), lambda i: (0, 0)),
        ],
        out_specs=pl.BlockSpec((512, 16), lambda i: (i, 0)),
        out_shape=jax.ShapeDtypeStruct((NP, 16), _f32),
    )(acc, h2p, dinv, b2, L1_W, L1_b, L2_W, L2_b)


def kernel(x, edge_index, W1, b1, W2, b2, L1_W, L1_b, L2_W, L2_b):
    x_pad = jnp.pad(x, ((0, NP - N), (0, 0)))
    pad_i = jnp.arange(EP - E, dtype=jnp.int32)
    pad_row = N + (pad_i % (NP - N))
    src = jnp.concatenate([edge_index[0], pad_row]).reshape(NW, CH, CL)
    dst = jnp.concatenate([edge_index[1], pad_row]).reshape(NW, CH, CL)
    ones_row = jnp.ones((CL,), _f32)

    degs = _sc_degree(dst, ones_row)
    degs = degs.reshape(2, NP, 1)

    h1p, dinv = _tc_layer1(x_pad, degs, W1)
    acc1 = _sc_gather_scatter(h1p, src, dst)
    h2p = _tc_layer2(acc1, h1p, dinv, b1.reshape(1, D), W2)
    acc2 = _sc_gather_scatter(h2p, src, dst)
    out = _tc_head(acc2, h2p, dinv, b2.reshape(1, D),
                   L1_W, L1_b.reshape(1, 4), L2_W, L2_b.reshape(1, 16))
    return out[:N]


# speedup vs baseline: 31.0425x; 1.0191x over previous
import functools

import jax
import jax.numpy as jnp
from jax import lax
from jax.experimental import pallas as pl
from jax.experimental.pallas import tpu as pltpu
from jax.experimental.pallas import tpu_sc as plsc

N = 10000
NP = 10240
D = 128
E = 320000
NW = 32
CH = 160
CL = 64
EP = NW * CH * CL
RB = NP // 16

_f32 = jnp.float32
_bf16 = jnp.bfloat16


def _sc_mesh():
    return plsc.VectorSubcoreMesh(core_axis_name="c", subcore_axis_name="s")


def _sc_degree(dst_idx, ones_row):
    @functools.partial(
        pl.kernel,
        out_type=jax.ShapeDtypeStruct((2, NP), _f32),
        mesh=_sc_mesh(),
        scratch_types=[
            pltpu.VMEM((CH, CL), jnp.int32),
            pltpu.VMEM((CL,), _f32),
            pltpu.VMEM((RB,), _f32),
            pltpu.VMEM_SHARED((NP,), _f32),
            pltpu.SemaphoreType.DMA,
            pltpu.SemaphoreType.DMA,
        ],
    )
    def k(dst_hbm, ones_hbm, out_hbm, idx_d, ones_v, zero_v, acc, s0, s1):
        cid = lax.axis_index("c")
        sid = lax.axis_index("s")
        wid = sid * 2 + cid
        pltpu.sync_copy(dst_hbm.at[wid], idx_d)
        pltpu.sync_copy(ones_hbm, ones_v)

        @pl.loop(0, RB // 16)
        def _(r):
            zero_v[pl.ds(r * 16, 16)] = jnp.zeros((16,), _f32)

        pltpu.sync_copy(zero_v, acc.at[pl.ds(sid * RB, RB)])
        plsc.subcore_barrier()

        @pl.loop(0, CH // 2)
        def _(j):
            d0 = pltpu.async_copy(ones_v, acc.at[idx_d.at[2 * j]], s0, add=True)
            d1 = pltpu.async_copy(ones_v, acc.at[idx_d.at[2 * j + 1]], s1, add=True)
            d0.wait()
            d1.wait()

        plsc.subcore_barrier()
        pltpu.sync_copy(acc.at[pl.ds(sid * RB, RB)],
                        out_hbm.at[cid, pl.ds(sid * RB, RB)])

    return k(dst_idx, ones_row)


def _sc_gather_scatter(table, src_idx, dst_idx):
    HC = CH // 4

    @functools.partial(
        pl.kernel,
        out_type=jax.ShapeDtypeStruct((2, NP, D), _f32),
        mesh=_sc_mesh(),
        scratch_types=[
            pltpu.VMEM((HC, CL), jnp.int32),
            pltpu.VMEM((HC, CL), jnp.int32),
            pltpu.VMEM((CL, D), _f32),
            pltpu.VMEM((CL, D), _f32),
            pltpu.VMEM((CL, D), _f32),
            pltpu.VMEM((CL, D), _f32),
            pltpu.VMEM_SHARED((NP, D), _f32),
            pltpu.SemaphoreType.DMA,
            pltpu.SemaphoreType.DMA,
            pltpu.SemaphoreType.DMA,
            pltpu.SemaphoreType.DMA,
            pltpu.SemaphoreType.DMA,
            pltpu.SemaphoreType.DMA,
            pltpu.SemaphoreType.DMA,
            pltpu.SemaphoreType.DMA,
        ],
    )
    def k(tab_hbm, src_hbm, dst_hbm, out_hbm,
          idx_s, idx_d, fb0, fb1, fb2, fb3, acc,
          sg0, sg1, sg2, sg3, ss0, ss1, ss2, ss3):
        cid = lax.axis_index("c")
        sid = lax.axis_index("s")
        wid = sid * 2 + cid
        bufs = ((fb0, sg0, ss0), (fb1, sg1, ss1),
                (fb2, sg2, ss2), (fb3, sg3, ss3))

        @pl.loop(0, CL)
        def _(r):
            for i in range(D // 16):
                fb0[r, pl.ds(i * 16, 16)] = jnp.zeros((16,), _f32)

        for z in range(RB // CL):
            pltpu.sync_copy(fb0, acc.at[pl.ds(sid * RB + z * CL, CL)])
        plsc.subcore_barrier()

        for h in range(4):
            pltpu.sync_copy(src_hbm.at[wid, pl.ds(h * HC, HC)], idx_s)
            pltpu.sync_copy(dst_hbm.at[wid, pl.ds(h * HC, HC)], idx_d)
            for p, (fb, sg, _) in enumerate(bufs):
                pltpu.async_copy(tab_hbm.at[idx_s.at[p]], fb, sg)

            @pl.loop(0, HC // 4)
            def _(j):
                for p, (fb, sg, ss) in enumerate(bufs):
                    pltpu.make_async_copy(tab_hbm.at[idx_s.at[0]], fb, sg).wait()
                    pltpu.async_copy(fb, acc.at[idx_d.at[4 * j + p]], ss, add=True)
                    pltpu.make_async_copy(fb, acc.at[idx_d.at[0]], ss).wait()

                    @pl.when(j < HC // 4 - 1)
                    def _():
                        pltpu.async_copy(tab_hbm.at[idx_s.at[4 * j + 4 + p]], fb, sg)

        plsc.subcore_barrier()
        pltpu.sync_copy(acc.at[pl.ds(sid * RB, RB)],
                        out_hbm.at[cid, pl.ds(sid * RB, RB)])

    return k(table, src_idx, dst_idx)


def _tc_layer1(x, degs, W1):
    def body(x_ref, d_ref, w_ref, h_ref, dinv_ref):
        din = lax.rsqrt(d_ref[0] + d_ref[1] + 1.0)
        h = jnp.dot(x_ref[...], w_ref[...], preferred_element_type=_f32)
        h_ref[...] = h * din
        dinv_ref[...] = din

    grid = NP // 512
    return pl.pallas_call(
        body,
        grid=(grid,),
        in_specs=[
            pl.BlockSpec((512, D), lambda i: (i, 0)),
            pl.BlockSpec((2, 512, 1), lambda i: (0, i, 0)),
            pl.BlockSpec((D, D), lambda i: (0, 0)),
        ],
        out_specs=[
            pl.BlockSpec((512, D), lambda i: (i, 0)),
            pl.BlockSpec((512, 1), lambda i: (i, 0)),
        ],
        out_shape=[
            jax.ShapeDtypeStruct((NP, D), _f32),
            jax.ShapeDtypeStruct((NP, 1), _f32),
        ],
    )(x, degs, W1)


def _tc_layer2(acc, h1p, dinv, b1, W2):
    def body(a_ref, h_ref, d_ref, b_ref, w_ref, o_ref):
        din = d_ref[...]
        s = (a_ref[0] + a_ref[1] + h_ref[...]) * din + b_ref[...]
        h2 = jnp.maximum(s, 0.0)
        o_ref[...] = jnp.dot(h2, w_ref[...], preferred_element_type=_f32) * din

    grid = NP // 512
    return pl.pallas_call(
        body,
        grid=(grid,),
        in_specs=[
            pl.BlockSpec((2, 512, D), lambda i: (0, i, 0)),
            pl.BlockSpec((512, D), lambda i: (i, 0)),
            pl.BlockSpec((512, 1), lambda i: (i, 0)),
            pl.BlockSpec((1, D), lambda i: (0, 0)),
            pl.BlockSpec((D, D), lambda i: (0, 0)),
        ],
        out_specs=pl.BlockSpec((512, D), lambda i: (i, 0)),
        out_shape=jax.ShapeDtypeStruct((NP, D), _f32),
    )(acc, h1p, dinv, b1, W2)


def _tc_head(acc, h2p, dinv, b2, L1_W, L1_b, L2_W, L2_b):
    def body(a_ref, h_ref, d_ref, b_ref, w1_ref, b1_ref, w2_ref, b2_ref, o_ref):
        s = (a_ref[0] + a_ref[1] + h_ref[...]) * d_ref[...] + b_ref[...]
        h3 = jnp.maximum(s, 0.0)
        t = jnp.dot(h3, w1_ref[...], preferred_element_type=_f32) + b1_ref[...]
        o_ref[...] = jnp.dot(t, w2_ref[...], preferred_element_type=_f32) + b2_ref[...]

    grid = NP // 512
    return pl.pallas_call(
        body,
        grid=(grid,),
        in_specs=[
            pl.BlockSpec((2, 512, D), lambda i: (0, i, 0)),
            pl.BlockSpec((512, D), lambda i: (i, 0)),
            pl.BlockSpec((512, 1), lambda i: (i, 0)),
            pl.BlockSpec((1, D), lambda i: (0, 0)),
            pl.BlockSpec((D, 4), lambda i: (0, 0)),
            pl.BlockSpec((1, 4), lambda i: (0, 0)),
            pl.BlockSpec((4, 16), lambda i: (0, 0)),
            pl.BlockSpec((1, 16), lambda i: (0, 0)),
        ],
        out_specs=pl.BlockSpec((512, 16), lambda i: (i, 0)),
        out_shape=jax.ShapeDtypeStruct((NP, 16), _f32),
    )(acc, h2p, dinv, b2, L1_W, L1_b, L2_W, L2_b)


def kernel(x, edge_index, W1, b1, W2, b2, L1_W, L1_b, L2_W, L2_b):
    x_pad = jnp.pad(x, ((0, NP - N), (0, 0)))
    pad_i = jnp.arange(EP - E, dtype=jnp.int32)
    pad_row = N + (pad_i % (NP - N))
    src = jnp.concatenate([edge_index[0], pad_row]).reshape(NW, CH, CL)
    dst = jnp.concatenate([edge_index[1], pad_row]).reshape(NW, CH, CL)
    ones_row = jnp.ones((CL,), _f32)

    degs = _sc_degree(dst, ones_row)
    degs = degs.reshape(2, NP, 1)

    h1p, dinv = _tc_layer1(x_pad, degs, W1)
    acc1 = _sc_gather_scatter(h1p, src, dst)
    h2p = _tc_layer2(acc1, h1p, dinv, b1.reshape(1, D), W2)
    acc2 = _sc_gather_scatter(h2p, src, dst)
    out = _tc_head(acc2, h2p, dinv, b2.reshape(1, D),
                   L1_W, L1_b.reshape(1, 4), L2_W, L2_b.reshape(1, 16))
    return out[:N]
